# baseline (speedup 1.0000x reference)
import functools

import jax
import jax.numpy as jnp
from jax import lax
from jax.experimental import pallas as pl
from jax.experimental.pallas import tpu as pltpu
from jax.experimental.pallas import tpu_sc as plsc

N = 10000
E = 320000
D = 128

NC, NS = 2, 16
NW = NC * NS
CHUNK = 64
NCHUNK = 160
EPT = NCHUNK * CHUNK
E_PAD = NW * EPT
ACC_ROWS = 10112
ZPT = ACC_ROWS // NS
Y_ROWS = 10240

_mesh = plsc.VectorSubcoreMesh(core_axis_name="c", subcore_axis_name="s")


@functools.partial(
    pl.kernel,
    out_type=jax.ShapeDtypeStruct((NC * ACC_ROWS,), jnp.float32),
    mesh=_mesh,
    scratch_types=[
        [pltpu.VMEM((8, CHUNK), jnp.int32)] * 4,
        pltpu.VMEM((CHUNK,), jnp.float32),
        pltpu.VMEM((ZPT + 8,), jnp.float32),
        pltpu.VMEM_SHARED((ACC_ROWS,), jnp.float32),
        [pltpu.SemaphoreType.DMA] * 4,
        [pltpu.SemaphoreType.DMA] * 4,
    ],
)
def _deg_kernel(dst_hbm, deg_out, didx, ones_v, stage_v, degs, isems, ssems):
    cid = lax.axis_index("c")
    sid = lax.axis_index("s")
    w = sid * NC + cid
    row0 = sid * ZPT
    zeros16 = jnp.zeros((16,), jnp.float32)
    ones16 = jnp.ones((16,), jnp.float32)
    for j in range(CHUNK // 16):
        ones_v[pl.ds(j * 16, 16)] = ones16

    def zbody(j, carry):
        stage_v[pl.ds(j * 16, 16)] = zeros16
        return carry

    lax.fori_loop(0, (ZPT + 8) // 16, zbody, 0)
    pltpu.sync_copy(stage_v.at[pl.ds(0, ZPT)], degs.at[pl.ds(row0, ZPT)])
    plsc.subcore_barrier()

    NSUP = NCHUNK // 8

    def start_fetch(q, s):
        pltpu.async_copy(dst_hbm.at[w, pl.ds(q * 8, 8)], didx[s], isems[s])

    def wait_fetch(s):
        pltpu.make_async_copy(dst_hbm.at[0, pl.ds(0, 8)], didx[s],
                              isems[s]).wait()

    def scatter_super(s):
        for j in range(8):
            pltpu.async_copy(ones_v, degs.at[didx[s].at[j]], ssems[s],
                             add=True)

    def drain_super(s):
        for j in range(8):
            pltpu.make_async_copy(ones_v, degs.at[didx[s].at[0]],
                                  ssems[s]).wait()

    for s in range(2):
        start_fetch(s, s)
    for q in range(2):
        wait_fetch(q)
        scatter_super(q)
        start_fetch(q + 2, q + 2)

    def group(g, carry):
        q0 = g * 4 + 2
        for k in range(4):
            p = (2 + k) % 4
            pf = k % 4
            wait_fetch(p)
            scatter_super(p)
            drain_super(pf)
            start_fetch(q0 + k + 2, pf)
        return carry

    lax.fori_loop(0, (NSUP - 4) // 4, group, 0)
    for q in range(NSUP - 2, NSUP):
        wait_fetch(q % 4)
        scatter_super(q % 4)
    for s in range(4):
        drain_super(s)
    plsc.subcore_barrier()
    pltpu.sync_copy(degs.at[pl.ds(row0, ZPT)], stage_v.at[pl.ds(0, ZPT)])
    pltpu.sync_copy(stage_v.at[pl.ds(0, ZPT)],
                    deg_out.at[pl.ds(cid * ACC_ROWS + row0, ZPT)])


@functools.partial(
    pl.kernel,
    out_type=jax.ShapeDtypeStruct((NC, ACC_ROWS, D), jnp.float32),
    mesh=_mesh,
    scratch_types=[
        [pltpu.VMEM((8, CHUNK), jnp.int32)] * 4,
        [pltpu.VMEM((8, CHUNK), jnp.int32)] * 4,
        [pltpu.VMEM((CHUNK, D), jnp.float32)] * 4,
        pltpu.VMEM_SHARED((ACC_ROWS, D), jnp.float32),
        [pltpu.SemaphoreType.DMA] * 4,
        [pltpu.SemaphoreType.DMA] * 4,
        [pltpu.SemaphoreType.DMA] * 4,
    ],
)
def _edge_kernel(y_hbm, src_hbm, dst_hbm, out_hbm,
                 sidx, didx, bufs, acc, gsems, ssems, isems):
    cid = lax.axis_index("c")
    sid = lax.axis_index("s")
    w = sid * NC + cid
    row0 = sid * ZPT
    zeros16 = jnp.zeros((16,), jnp.float32)
    buf0 = bufs[0]

    def zbody(r, carry):
        for j in range(D // 16):
            buf0[r, pl.ds(j * 16, 16)] = zeros16
        return carry

    lax.fori_loop(0, CHUNK, zbody, 0)
    _rem = ZPT % CHUNK
    for k in range(ZPT // CHUNK):
        pltpu.sync_copy(buf0, acc.at[pl.ds(row0 + k * CHUNK, CHUNK)])
    if _rem:
        pltpu.sync_copy(buf0.at[pl.ds(0, _rem)],
                        acc.at[pl.ds(row0 + (ZPT // CHUNK) * CHUNK, _rem)])
    plsc.subcore_barrier()

    NSUP = NCHUNK // 8

    def start_fetch(q, s):
        pltpu.async_copy(src_hbm.at[w, pl.ds(q * 8, 8)], sidx[s], isems[s])
        pltpu.async_copy(dst_hbm.at[w, pl.ds(q * 8, 8)], didx[s], isems[s])

    def wait_fetch(s):
        pltpu.make_async_copy(src_hbm.at[0, pl.ds(0, 8)], sidx[s],
                              isems[s]).wait()
        pltpu.make_async_copy(dst_hbm.at[0, pl.ds(0, 8)], didx[s],
                              isems[s]).wait()

    def start_gather(b, p, r):
        pltpu.async_copy(y_hbm.at[sidx[p].at[r]], bufs[b], gsems[b])

    def wait_gather(b):
        pltpu.make_async_copy(y_hbm.at[sidx[0].at[0]], bufs[b],
                              gsems[b]).wait()

    def start_scatter(b, p, r):
        pltpu.async_copy(bufs[b], acc.at[didx[p].at[r]], ssems[b], add=True)

    def wait_scatter(b):
        pltpu.make_async_copy(bufs[0], acc.at[didx[0].at[0]],
                              ssems[b]).wait()

    def super_body(p, pn, first_super=False, next_fetch_slot=None,
                   last_super=False):
        for k in range(8):
            b0 = k % 4
            b2 = (k + 2) % 4
            wait_gather(b0)
            start_scatter(b0, p, k)
            if not (first_super and k < 2):
                wait_scatter(b2)
            if k == 7 and next_fetch_slot is not None:
                wait_fetch(next_fetch_slot)
            if not (last_super and k == 7):
                if k < 7:
                    start_gather((k + 1) % 4, p, k + 1)
                else:
                    start_gather(0, pn, 0)

    start_fetch(0, 0)
    start_fetch(1, 1)
    wait_fetch(0)
    start_gather(0, 0, 0)
    start_fetch(2, 2)
    super_body(0, 1, first_super=True, next_fetch_slot=1)
    start_fetch(3, 3)
    super_body(1, 2, next_fetch_slot=2)

    def group(g, carry):
        for m in range(4):
            p = (2 + m) % 4
            pn = (3 + m) % 4
            q = g * 4 + 2 + m
            start_fetch(q + 2, m % 4)
            super_body(p, pn, next_fetch_slot=pn)
        return carry

    lax.fori_loop(0, (NSUP - 4) // 4, group, 0)
    super_body(2, 3, next_fetch_slot=3)
    super_body(3, None, last_super=True)
    wait_scatter(2)
    wait_scatter(3)
    plsc.subcore_barrier()
    for k in range(ZPT // CHUNK):
        pltpu.sync_copy(acc.at[pl.ds(row0 + k * CHUNK, CHUNK)], buf0)
        pltpu.sync_copy(buf0, out_hbm.at[cid, pl.ds(row0 + k * CHUNK, CHUNK)])
    if _rem:
        _off = row0 + (ZPT // CHUNK) * CHUNK
        pltpu.sync_copy(acc.at[pl.ds(_off, _rem)], buf0.at[pl.ds(0, _rem)])
        pltpu.sync_copy(buf0.at[pl.ds(0, _rem)],
                        out_hbm.at[cid, pl.ds(_off, _rem)])


_BLK = 1000


def _mm_body(x_ref, w_ref, d0_ref, d1_ref, y_ref, dinv_ref):
    deg = d0_ref[...] + d1_ref[...] + 1.0
    dinv = lax.rsqrt(deg)
    xw = jnp.dot(x_ref[...], w_ref[...], preferred_element_type=jnp.float32)
    y_ref[...] = xw * dinv
    dinv_ref[...] = dinv


def _fin_body(a0_ref, a1_ref, y_ref, dinv_ref, b_ref, g_ref, be_ref, o_ref):
    s = a0_ref[0] + a1_ref[0] + y_ref[...]
    pre = s * dinv_ref[...] + b_ref[...]
    mu = jnp.mean(pre, axis=-1, keepdims=True)
    ctr = pre - mu
    var = jnp.mean(ctr * ctr, axis=-1, keepdims=True)
    h = ctr * lax.rsqrt(var + 1e-5) * g_ref[...] + be_ref[...]
    o_ref[...] = jnp.maximum(h, 0.0)


def kernel(x, edge_index, W, b, ln_gamma, ln_beta):
    ei = edge_index.astype(jnp.int32)
    pad = N + (jnp.arange(E_PAD - E, dtype=jnp.int32) % 8)
    src_p = jnp.concatenate([ei[0], pad]).reshape(NW, NCHUNK, CHUNK)
    dst_p = jnp.concatenate([ei[1], pad]).reshape(NW, NCHUNK, CHUNK)

    deg_parts = _deg_kernel(dst_p)
    d0 = deg_parts[:N].reshape(N, 1)
    d1 = deg_parts[ACC_ROWS:ACC_ROWS + N].reshape(N, 1)

    _YB = Y_ROWS // 10
    y, dinv = pl.pallas_call(
        _mm_body,
        grid=(10,),
        in_specs=[
            pl.BlockSpec((_YB, D), lambda i: (i, 0)),
            pl.BlockSpec((D, D), lambda i: (0, 0)),
            pl.BlockSpec((_YB, 1), lambda i: (i, 0)),
            pl.BlockSpec((_YB, 1), lambda i: (i, 0)),
        ],
        out_specs=[
            pl.BlockSpec((_YB, D), lambda i: (i, 0)),
            pl.BlockSpec((_YB, 1), lambda i: (i, 0)),
        ],
        out_shape=[
            jax.ShapeDtypeStruct((Y_ROWS, D), jnp.float32),
            jax.ShapeDtypeStruct((Y_ROWS, 1), jnp.float32),
        ],
    )(x, W, d0, d1)

    acc_parts = _edge_kernel(y, src_p, dst_p)

    out = pl.pallas_call(
        _fin_body,
        grid=(N // _BLK,),
        in_specs=[
            pl.BlockSpec((1, _BLK, D), lambda i: (0, i, 0)),
            pl.BlockSpec((1, _BLK, D), lambda i: (1, i, 0)),
            pl.BlockSpec((_BLK, D), lambda i: (i, 0)),
            pl.BlockSpec((_BLK, 1), lambda i: (i, 0)),
            pl.BlockSpec((1, D), lambda i: (0, 0)),
            pl.BlockSpec((1, D), lambda i: (0, 0)),
            pl.BlockSpec((1, D), lambda i: (0, 0)),
        ],
        out_specs=pl.BlockSpec((_BLK, D), lambda i: (i, 0)),
        out_shape=jax.ShapeDtypeStruct((N, D), jnp.float32),
    )(acc_parts, acc_parts, y, dinv,
      b.reshape(1, D), ln_gamma.reshape(1, D), ln_beta.reshape(1, D))
    return out


# speedup vs baseline: 1.2475x; 1.2475x over previous
import functools

import jax
import jax.numpy as jnp
from jax import lax
from jax.experimental import pallas as pl
from jax.experimental.pallas import tpu as pltpu
from jax.experimental.pallas import tpu_sc as plsc

N = 10000
E = 320000
D = 128

NC, NS = 2, 16
NW = NC * NS
CHUNK = 128
NCHUNK = 80
EPT = NCHUNK * CHUNK
E_PAD = NW * EPT
ACC_ROWS = 10112
ZPT = ACC_ROWS // NS
Y_ROWS = 10240

_mesh = plsc.VectorSubcoreMesh(core_axis_name="c", subcore_axis_name="s")


@functools.partial(
    pl.kernel,
    out_type=jax.ShapeDtypeStruct((NC * ACC_ROWS,), jnp.float32),
    mesh=_mesh,
    scratch_types=[
        [pltpu.VMEM((4, CHUNK), jnp.int32)] * 4,
        pltpu.VMEM((CHUNK,), jnp.float32),
        pltpu.VMEM((ZPT + 8,), jnp.float32),
        pltpu.VMEM_SHARED((ACC_ROWS,), jnp.float32),
        [pltpu.SemaphoreType.DMA] * 4,
        [pltpu.SemaphoreType.DMA] * 4,
    ],
)
def _deg_kernel(dst_hbm, deg_out, didx, ones_v, stage_v, degs, isems, ssems):
    cid = lax.axis_index("c")
    sid = lax.axis_index("s")
    w = sid * NC + cid
    row0 = sid * ZPT
    zeros16 = jnp.zeros((16,), jnp.float32)
    ones16 = jnp.ones((16,), jnp.float32)
    for j in range(CHUNK // 16):
        ones_v[pl.ds(j * 16, 16)] = ones16

    def zbody(j, carry):
        stage_v[pl.ds(j * 16, 16)] = zeros16
        return carry

    lax.fori_loop(0, (ZPT + 8) // 16, zbody, 0)
    pltpu.sync_copy(stage_v.at[pl.ds(0, ZPT)], degs.at[pl.ds(row0, ZPT)])
    plsc.subcore_barrier()

    NSUP = NCHUNK // 4

    def start_fetch(q, s):
        pltpu.async_copy(dst_hbm.at[w, pl.ds(q * 4, 4)], didx[s], isems[s])

    def wait_fetch(s):
        pltpu.make_async_copy(dst_hbm.at[0, pl.ds(0, 4)], didx[s],
                              isems[s]).wait()

    def scatter_super(s):
        for j in range(4):
            pltpu.async_copy(ones_v, degs.at[didx[s].at[j]], ssems[s],
                             add=True)

    def drain_super(s):
        for j in range(4):
            pltpu.make_async_copy(ones_v, degs.at[didx[s].at[0]],
                                  ssems[s]).wait()

    for s in range(2):
        start_fetch(s, s)
    for q in range(2):
        wait_fetch(q)
        scatter_super(q)
        start_fetch(q + 2, q + 2)

    def group(g, carry):
        q0 = g * 4 + 2
        for k in range(4):
            p = (2 + k) % 4
            pf = k % 4
            wait_fetch(p)
            scatter_super(p)
            drain_super(pf)
            start_fetch(q0 + k + 2, pf)
        return carry

    lax.fori_loop(0, (NSUP - 4) // 4, group, 0)
    for q in range(NSUP - 2, NSUP):
        wait_fetch(q % 4)
        scatter_super(q % 4)
    for s in range(4):
        drain_super(s)
    plsc.subcore_barrier()
    pltpu.sync_copy(degs.at[pl.ds(row0, ZPT)], stage_v.at[pl.ds(0, ZPT)])
    pltpu.sync_copy(stage_v.at[pl.ds(0, ZPT)],
                    deg_out.at[pl.ds(cid * ACC_ROWS + row0, ZPT)])


@functools.partial(
    pl.kernel,
    out_type=jax.ShapeDtypeStruct((NC, ACC_ROWS, D), jnp.float32),
    mesh=_mesh,
    scratch_types=[
        [pltpu.VMEM((CHUNK,), jnp.int32)] * 4,
        [pltpu.VMEM((CHUNK,), jnp.int32)] * 4,
        [pltpu.VMEM((CHUNK, D), jnp.float32)] * 2,
        pltpu.VMEM_SHARED((ACC_ROWS, D), jnp.float32),
        [pltpu.SemaphoreType.DMA] * 2,
        [pltpu.SemaphoreType.DMA] * 2,
        [pltpu.SemaphoreType.DMA] * 4,
    ],
)
def _edge_kernel(y_hbm, src_hbm, dst_hbm, out_hbm,
                 sidx, didx, bufs, acc, gsems, ssems, isems):
    cid = lax.axis_index("c")
    sid = lax.axis_index("s")
    w = sid * NC + cid
    row0 = sid * ZPT
    zeros16 = jnp.zeros((16,), jnp.float32)
    buf0 = bufs[0]

    def zbody(r, carry):
        for j in range(D // 16):
            buf0[r, pl.ds(j * 16, 16)] = zeros16
        return carry

    lax.fori_loop(0, CHUNK, zbody, 0)
    _rem = ZPT % CHUNK
    for k in range(ZPT // CHUNK):
        pltpu.sync_copy(buf0, acc.at[pl.ds(row0 + k * CHUNK, CHUNK)])
    if _rem:
        pltpu.sync_copy(buf0.at[pl.ds(0, _rem)],
                        acc.at[pl.ds(row0 + (ZPT // CHUNK) * CHUNK, _rem)])
    plsc.subcore_barrier()

    def start_fetch(c, p):
        pltpu.async_copy(src_hbm.at[w, c], sidx[p], isems[p])
        pltpu.async_copy(dst_hbm.at[w, c], didx[p], isems[p])

    def wait_fetch(p):
        pltpu.make_async_copy(src_hbm.at[0, 0], sidx[p], isems[p]).wait()
        pltpu.make_async_copy(dst_hbm.at[0, 0], didx[p], isems[p]).wait()

    def start_gather(b, p):
        pltpu.async_copy(y_hbm.at[sidx[p]], bufs[b], gsems[b])

    def wait_gather(b):
        pltpu.make_async_copy(y_hbm.at[sidx[0]], bufs[b], gsems[b]).wait()

    def start_scatter(b, p):
        pltpu.async_copy(bufs[b], acc.at[didx[p]], ssems[b], add=True)

    def wait_scatter(b):
        pltpu.make_async_copy(bufs[0], acc.at[didx[0]], ssems[b]).wait()

    def body(c, k, fetch=True):
        b = k % 2
        p = k % 4
        wait_gather(b)
        start_scatter(b, p)
        wait_scatter(1 - b)
        if fetch:
            start_fetch(c + 3, (k + 3) % 4)
        wait_fetch((k + 1) % 4)
        start_gather(1 - b, (k + 1) % 4)

    for s in range(4):
        start_fetch(s, s)
    wait_fetch(0)
    start_gather(0, 0)
    wait_gather(0)
    start_scatter(0, 0)
    wait_fetch(1)
    start_gather(1, 1)
    wait_gather(1)
    start_scatter(1, 1)
    wait_scatter(0)
    start_fetch(4, 0)
    wait_fetch(2)
    start_gather(0, 2)

    def group(g, carry):
        c0 = g * 4 + 2
        for k in range(4):
            body(c0 + k, 2 + k)
        return carry

    lax.fori_loop(0, (NCHUNK - 8) // 4, group, 0)
    for kk in range(5):
        c = NCHUNK - 6 + kk
        body(c, 2 + kk, fetch=(c + 3 < NCHUNK))
    wait_gather(1)
    start_scatter(1, 3)
    wait_scatter(0)
    wait_scatter(1)
    plsc.subcore_barrier()
    for k in range(ZPT // CHUNK):
        pltpu.sync_copy(acc.at[pl.ds(row0 + k * CHUNK, CHUNK)], buf0)
        pltpu.sync_copy(buf0, out_hbm.at[cid, pl.ds(row0 + k * CHUNK, CHUNK)])
    if _rem:
        _off = row0 + (ZPT // CHUNK) * CHUNK
        pltpu.sync_copy(acc.at[pl.ds(_off, _rem)], buf0.at[pl.ds(0, _rem)])
        pltpu.sync_copy(buf0.at[pl.ds(0, _rem)],
                        out_hbm.at[cid, pl.ds(_off, _rem)])


_BLK = 1000


def _mm_body(x_ref, w_ref, d0_ref, d1_ref, y_ref, dinv_ref):
    deg = d0_ref[...] + d1_ref[...] + 1.0
    dinv = lax.rsqrt(deg)
    xw = jnp.dot(x_ref[...], w_ref[...], preferred_element_type=jnp.float32)
    y_ref[...] = xw * dinv
    dinv_ref[...] = dinv


def _fin_body(a0_ref, a1_ref, y_ref, dinv_ref, b_ref, g_ref, be_ref, o_ref):
    s = a0_ref[0] + a1_ref[0] + y_ref[...]
    pre = s * dinv_ref[...] + b_ref[...]
    mu = jnp.mean(pre, axis=-1, keepdims=True)
    ctr = pre - mu
    var = jnp.mean(ctr * ctr, axis=-1, keepdims=True)
    h = ctr * lax.rsqrt(var + 1e-5) * g_ref[...] + be_ref[...]
    o_ref[...] = jnp.maximum(h, 0.0)


def kernel(x, edge_index, W, b, ln_gamma, ln_beta):
    ei = edge_index.astype(jnp.int32)
    pad = N + (jnp.arange(E_PAD - E, dtype=jnp.int32) % 8)
    src_p = jnp.concatenate([ei[0], pad]).reshape(NW, NCHUNK, CHUNK)
    dst_p = jnp.concatenate([ei[1], pad]).reshape(NW, NCHUNK, CHUNK)

    deg_parts = _deg_kernel(dst_p)
    d0 = deg_parts[:N].reshape(N, 1)
    d1 = deg_parts[ACC_ROWS:ACC_ROWS + N].reshape(N, 1)

    _YB = Y_ROWS // 10
    y, dinv = pl.pallas_call(
        _mm_body,
        grid=(10,),
        in_specs=[
            pl.BlockSpec((_YB, D), lambda i: (i, 0)),
            pl.BlockSpec((D, D), lambda i: (0, 0)),
            pl.BlockSpec((_YB, 1), lambda i: (i, 0)),
            pl.BlockSpec((_YB, 1), lambda i: (i, 0)),
        ],
        out_specs=[
            pl.BlockSpec((_YB, D), lambda i: (i, 0)),
            pl.BlockSpec((_YB, 1), lambda i: (i, 0)),
        ],
        out_shape=[
            jax.ShapeDtypeStruct((Y_ROWS, D), jnp.float32),
            jax.ShapeDtypeStruct((Y_ROWS, 1), jnp.float32),
        ],
    )(x, W, d0, d1)

    acc_parts = _edge_kernel(y, src_p, dst_p)

    out = pl.pallas_call(
        _fin_body,
        grid=(N // _BLK,),
        in_specs=[
            pl.BlockSpec((1, _BLK, D), lambda i: (0, i, 0)),
            pl.BlockSpec((1, _BLK, D), lambda i: (1, i, 0)),
            pl.BlockSpec((_BLK, D), lambda i: (i, 0)),
            pl.BlockSpec((_BLK, 1), lambda i: (i, 0)),
            pl.BlockSpec((1, D), lambda i: (0, 0)),
            pl.BlockSpec((1, D), lambda i: (0, 0)),
            pl.BlockSpec((1, D), lambda i: (0, 0)),
        ],
        out_specs=pl.BlockSpec((_BLK, D), lambda i: (i, 0)),
        out_shape=jax.ShapeDtypeStruct((N, D), jnp.float32),
    )(acc_parts, acc_parts, y, dinv,
      b.reshape(1, D), ln_gamma.reshape(1, D), ln_beta.reshape(1, D))
    return out


# speedup vs baseline: 1.3553x; 1.0864x over previous
import functools

import jax
import jax.numpy as jnp
from jax import lax
from jax.experimental import pallas as pl
from jax.experimental.pallas import tpu as pltpu
from jax.experimental.pallas import tpu_sc as plsc

N = 10000
E = 320000
D = 128

NC, NS = 2, 16
NW = NC * NS
CHUNK = 128
EPT = E // NW
NCHUNK = EPT // CHUNK
TAIL = EPT - NCHUNK * CHUNK
ACC_ROWS = 10112
ZPT = ACC_ROWS // NS

_mesh = plsc.VectorSubcoreMesh(core_axis_name="c", subcore_axis_name="s")


@functools.partial(
    pl.kernel,
    out_type=jax.ShapeDtypeStruct((NC * ACC_ROWS,), jnp.float32),
    mesh=_mesh,
    scratch_types=[
        [pltpu.VMEM((CHUNK,), jnp.int32)] * 4,
        pltpu.VMEM((TAIL,), jnp.int32),
        pltpu.VMEM((CHUNK,), jnp.float32),
        pltpu.VMEM((ZPT + 8,), jnp.float32),
        pltpu.VMEM_SHARED((ACC_ROWS,), jnp.float32),
        [pltpu.SemaphoreType.DMA] * 4,
        [pltpu.SemaphoreType.DMA] * 4,
        pltpu.SemaphoreType.DMA,
    ],
)
def _deg_kernel(dst_hbm, deg_out, didx, tidx, ones_v, stage_v, degs,
                isems, ssems, tsem):
    cid = lax.axis_index("c")
    sid = lax.axis_index("s")
    w = sid * NC + cid
    base = w * EPT
    row0 = sid * ZPT
    zeros16 = jnp.zeros((16,), jnp.float32)
    ones16 = jnp.ones((16,), jnp.float32)
    for j in range(CHUNK // 16):
        ones_v[pl.ds(j * 16, 16)] = ones16

    def zbody(j, carry):
        stage_v[pl.ds(j * 16, 16)] = zeros16
        return carry

    lax.fori_loop(0, (ZPT + 8) // 16, zbody, 0)
    pltpu.sync_copy(stage_v.at[pl.ds(0, ZPT)], degs.at[pl.ds(row0, ZPT)])
    plsc.subcore_barrier()

    def start_fetch(c, p):
        pltpu.async_copy(dst_hbm.at[pl.ds(base + c * CHUNK, CHUNK)],
                         didx[p], isems[p])

    def wait_fetch(p):
        pltpu.make_async_copy(dst_hbm.at[pl.ds(0, CHUNK)], didx[p],
                              isems[p]).wait()

    def start_scatter(p):
        pltpu.async_copy(ones_v, degs.at[didx[p]], ssems[p], add=True)

    def wait_scatter(p):
        pltpu.make_async_copy(ones_v, degs.at[didx[0]], ssems[p]).wait()

    pltpu.async_copy(dst_hbm.at[pl.ds(base + NCHUNK * CHUNK, TAIL)],
                     tidx, tsem)
    for s in range(4):
        start_fetch(s, s)
    for c in range(2):
        wait_fetch(c)
        start_scatter(c)
    for c in range(2, 6):
        wait_fetch(c % 4)
        start_scatter(c % 4)
        wait_scatter((c + 2) % 4)
        start_fetch(c + 2, (c + 2) % 4)

    def group(g, carry):
        c0 = g * 4 + 6
        for k in range(4):
            p = (2 + k) % 4
            wait_fetch(p)
            start_scatter(p)
            wait_scatter(k % 4)
            start_fetch(c0 + k + 2, k % 4)
        return carry

    lax.fori_loop(0, (NCHUNK - 10) // 4, group, 0)
    for c in range(NCHUNK - 4, NCHUNK):
        p = c % 4
        wait_fetch(p)
        start_scatter(p)
        wait_scatter((c + 2) % 4)
        if c + 2 < NCHUNK:
            start_fetch(c + 2, (c + 2) % 4)
    wait_scatter((NCHUNK - 2) % 4)
    wait_scatter((NCHUNK - 1) % 4)
    pltpu.make_async_copy(dst_hbm.at[pl.ds(0, TAIL)], tidx, tsem).wait()
    pltpu.sync_copy(ones_v.at[pl.ds(0, TAIL)], degs.at[tidx], add=True)
    plsc.subcore_barrier()
    pltpu.sync_copy(degs.at[pl.ds(row0, ZPT)], stage_v.at[pl.ds(0, ZPT)])
    pltpu.sync_copy(stage_v.at[pl.ds(0, ZPT)],
                    deg_out.at[pl.ds(cid * ACC_ROWS + row0, ZPT)])


@functools.partial(
    pl.kernel,
    out_type=jax.ShapeDtypeStruct((NC, ACC_ROWS, D), jnp.float32),
    mesh=_mesh,
    scratch_types=[
        [pltpu.VMEM((CHUNK,), jnp.int32)] * 4,
        [pltpu.VMEM((CHUNK,), jnp.int32)] * 4,
        pltpu.VMEM((TAIL,), jnp.int32),
        pltpu.VMEM((TAIL,), jnp.int32),
        [pltpu.VMEM((CHUNK, D), jnp.float32)] * 2,
        pltpu.VMEM((TAIL, D), jnp.float32),
        pltpu.VMEM_SHARED((ACC_ROWS, D), jnp.float32),
        [pltpu.SemaphoreType.DMA] * 2,
        [pltpu.SemaphoreType.DMA] * 2,
        [pltpu.SemaphoreType.DMA] * 4,
        pltpu.SemaphoreType.DMA,
    ],
)
def _edge_kernel(y_hbm, src_hbm, dst_hbm, out_hbm,
                 sidx, didx, tsidx, tdidx, bufs, tbuf, acc,
                 gsems, ssems, isems, tsem):
    cid = lax.axis_index("c")
    sid = lax.axis_index("s")
    w = sid * NC + cid
    base = w * EPT
    row0 = sid * ZPT
    zeros16 = jnp.zeros((16,), jnp.float32)
    buf0 = bufs[0]

    def zbody(r, carry):
        for j in range(D // 16):
            buf0[r, pl.ds(j * 16, 16)] = zeros16
        return carry

    lax.fori_loop(0, CHUNK, zbody, 0)
    _rem = ZPT % CHUNK
    for k in range(ZPT // CHUNK):
        pltpu.sync_copy(buf0, acc.at[pl.ds(row0 + k * CHUNK, CHUNK)])
    if _rem:
        pltpu.sync_copy(buf0.at[pl.ds(0, _rem)],
                        acc.at[pl.ds(row0 + (ZPT // CHUNK) * CHUNK, _rem)])
    plsc.subcore_barrier()

    def start_fetch(c, p):
        pltpu.async_copy(src_hbm.at[pl.ds(base + c * CHUNK, CHUNK)],
                         sidx[p], isems[p])
        pltpu.async_copy(dst_hbm.at[pl.ds(base + c * CHUNK, CHUNK)],
                         didx[p], isems[p])

    def wait_fetch(p):
        pltpu.make_async_copy(src_hbm.at[pl.ds(0, CHUNK)], sidx[p],
                              isems[p]).wait()
        pltpu.make_async_copy(dst_hbm.at[pl.ds(0, CHUNK)], didx[p],
                              isems[p]).wait()

    def start_gather(b, p):
        pltpu.async_copy(y_hbm.at[sidx[p]], bufs[b], gsems[b])

    def wait_gather(b):
        pltpu.make_async_copy(y_hbm.at[sidx[0]], bufs[b], gsems[b]).wait()

    def start_scatter(b, p):
        pltpu.async_copy(bufs[b], acc.at[didx[p]], ssems[b], add=True)

    def wait_scatter(b):
        pltpu.make_async_copy(bufs[0], acc.at[didx[0]], ssems[b]).wait()

    def body(c, k, fetch=True):
        b = k % 2
        p = k % 4
        wait_gather(b)
        start_scatter(b, p)
        wait_scatter(1 - b)
        if fetch:
            start_fetch(c + 3, (k + 3) % 4)
        wait_fetch((k + 1) % 4)
        start_gather(1 - b, (k + 1) % 4)

    pltpu.async_copy(src_hbm.at[pl.ds(base + NCHUNK * CHUNK, TAIL)],
                     tsidx, tsem)
    pltpu.async_copy(dst_hbm.at[pl.ds(base + NCHUNK * CHUNK, TAIL)],
                     tdidx, tsem)
    for s in range(4):
        start_fetch(s, s)
    wait_fetch(0)
    start_gather(0, 0)
    wait_gather(0)
    start_scatter(0, 0)
    wait_fetch(1)
    start_gather(1, 1)
    wait_gather(1)
    start_scatter(1, 1)
    wait_scatter(0)
    start_fetch(4, 0)
    wait_fetch(2)
    start_gather(0, 2)

    def group(g, carry):
        c0 = g * 4 + 2
        for k in range(4):
            body(c0 + k, 2 + k)
        return carry

    lax.fori_loop(0, (NCHUNK - 10) // 4, group, 0)
    for kk in range(7):
        c = NCHUNK - 8 + kk
        body(c, c, fetch=(c + 3 < NCHUNK))
    wait_gather((NCHUNK - 1) % 2)
    start_scatter((NCHUNK - 1) % 2, (NCHUNK - 1) % 4)
    wait_scatter((NCHUNK - 2) % 2)
    wait_scatter((NCHUNK - 1) % 2)
    pltpu.make_async_copy(src_hbm.at[pl.ds(0, TAIL)], tsidx, tsem).wait()
    pltpu.make_async_copy(dst_hbm.at[pl.ds(0, TAIL)], tdidx, tsem).wait()
    pltpu.async_copy(y_hbm.at[tsidx], tbuf, gsems[0])
    pltpu.make_async_copy(y_hbm.at[tsidx], tbuf, gsems[0]).wait()
    pltpu.sync_copy(tbuf, acc.at[tdidx], add=True)
    plsc.subcore_barrier()
    for k in range(ZPT // CHUNK):
        pltpu.sync_copy(acc.at[pl.ds(row0 + k * CHUNK, CHUNK)], buf0)
        pltpu.sync_copy(buf0, out_hbm.at[cid, pl.ds(row0 + k * CHUNK, CHUNK)])
    if _rem:
        _off = row0 + (ZPT // CHUNK) * CHUNK
        pltpu.sync_copy(acc.at[pl.ds(_off, _rem)], buf0.at[pl.ds(0, _rem)])
        pltpu.sync_copy(buf0.at[pl.ds(0, _rem)],
                        out_hbm.at[cid, pl.ds(_off, _rem)])


_BLK = 1000


def _mm_body(x_ref, w_ref, ds_ref, y_ref, dinv_ref):
    dinv = lax.rsqrt(ds_ref[...] + 1.0)
    xw = jnp.dot(x_ref[...], w_ref[...], preferred_element_type=jnp.float32)
    y_ref[...] = xw * dinv
    dinv_ref[...] = dinv


def _fin_body(a0_ref, a1_ref, y_ref, dinv_ref, b_ref, g_ref, be_ref, o_ref):
    s = a0_ref[0] + a1_ref[0] + y_ref[...]
    pre = s * dinv_ref[...] + b_ref[...]
    mu = jnp.mean(pre, axis=-1, keepdims=True)
    ctr = pre - mu
    var = jnp.mean(ctr * ctr, axis=-1, keepdims=True)
    h = ctr * lax.rsqrt(var + 1e-5) * g_ref[...] + be_ref[...]
    o_ref[...] = jnp.maximum(h, 0.0)


def kernel(x, edge_index, W, b, ln_gamma, ln_beta):
    ei = edge_index.astype(jnp.int32)
    src1 = ei[0]
    dst1 = ei[1]

    deg_parts = _deg_kernel(dst1)
    degsum = (deg_parts[:N] + deg_parts[ACC_ROWS:ACC_ROWS + N]).reshape(N, 1)

    y, dinv = pl.pallas_call(
        _mm_body,
        grid=(N // _BLK,),
        in_specs=[
            pl.BlockSpec((_BLK, D), lambda i: (i, 0)),
            pl.BlockSpec((D, D), lambda i: (0, 0)),
            pl.BlockSpec((_BLK, 1), lambda i: (i, 0)),
        ],
        out_specs=[
            pl.BlockSpec((_BLK, D), lambda i: (i, 0)),
            pl.BlockSpec((_BLK, 1), lambda i: (i, 0)),
        ],
        out_shape=[
            jax.ShapeDtypeStruct((N, D), jnp.float32),
            jax.ShapeDtypeStruct((N, 1), jnp.float32),
        ],
    )(x, W, degsum)

    acc_parts = _edge_kernel(y, src1, dst1)

    out = pl.pallas_call(
        _fin_body,
        grid=(N // _BLK,),
        in_specs=[
            pl.BlockSpec((1, _BLK, D), lambda i: (0, i, 0)),
            pl.BlockSpec((1, _BLK, D), lambda i: (1, i, 0)),
            pl.BlockSpec((_BLK, D), lambda i: (i, 0)),
            pl.BlockSpec((_BLK, 1), lambda i: (i, 0)),
            pl.BlockSpec((1, D), lambda i: (0, 0)),
            pl.BlockSpec((1, D), lambda i: (0, 0)),
            pl.BlockSpec((1, D), lambda i: (0, 0)),
        ],
        out_specs=pl.BlockSpec((_BLK, D), lambda i: (i, 0)),
        out_shape=jax.ShapeDtypeStruct((N, D), jnp.float32),
    )(acc_parts, acc_parts, y, dinv,
      b.reshape(1, D), ln_gamma.reshape(1, D), ln_beta.reshape(1, D))
    return out


# speedup vs baseline: 1.3650x; 1.0071x over previous
import functools

import jax
import jax.numpy as jnp
from jax import lax
from jax.experimental import pallas as pl
from jax.experimental.pallas import tpu as pltpu
from jax.experimental.pallas import tpu_sc as plsc

N = 10000
E = 320000
D = 128

NC, NS = 2, 16
NW = NC * NS
CHUNK = 128
EPT = E // NW
NCHUNK = EPT // CHUNK
TAIL = EPT - NCHUNK * CHUNK
ACC_ROWS = 10112
ZPT = ACC_ROWS // NS

_mesh = plsc.VectorSubcoreMesh(core_axis_name="c", subcore_axis_name="s")


@functools.partial(
    pl.kernel,
    out_type=jax.ShapeDtypeStruct((NC * ACC_ROWS,), jnp.float32),
    mesh=_mesh,
    scratch_types=[
        [pltpu.VMEM((CHUNK,), jnp.int32)] * 4,
        pltpu.VMEM((TAIL,), jnp.int32),
        pltpu.VMEM((CHUNK,), jnp.float32),
        pltpu.VMEM((ZPT + 8,), jnp.float32),
        pltpu.VMEM_SHARED((ACC_ROWS,), jnp.float32),
        [pltpu.SemaphoreType.DMA] * 4,
        [pltpu.SemaphoreType.DMA] * 4,
        pltpu.SemaphoreType.DMA,
    ],
)
def _deg_kernel(dst_hbm, deg_out, didx, tidx, ones_v, stage_v, degs,
                isems, ssems, tsem):
    cid = lax.axis_index("c")
    sid = lax.axis_index("s")
    w = sid * NC + cid
    base = w * EPT
    row0 = sid * ZPT

    def start_fetch(c, p):
        pltpu.async_copy(dst_hbm.at[pl.ds(base + c * CHUNK, CHUNK)],
                         didx[p], isems[p])

    def wait_fetch(p):
        pltpu.make_async_copy(dst_hbm.at[pl.ds(0, CHUNK)], didx[p],
                              isems[p]).wait()

    def start_scatter(p):
        pltpu.async_copy(ones_v, degs.at[didx[p]], ssems[p], add=True)

    def wait_scatter(p):
        pltpu.make_async_copy(ones_v, degs.at[didx[0]], ssems[p]).wait()

    pltpu.async_copy(dst_hbm.at[pl.ds(base + NCHUNK * CHUNK, TAIL)],
                     tidx, tsem)
    for s in range(4):
        start_fetch(s, s)
    zeros16 = jnp.zeros((16,), jnp.float32)
    ones16 = jnp.ones((16,), jnp.float32)
    for j in range(CHUNK // 16):
        ones_v[pl.ds(j * 16, 16)] = ones16

    def zbody(j, carry):
        stage_v[pl.ds(j * 16, 16)] = zeros16
        return carry

    lax.fori_loop(0, (ZPT + 8) // 16, zbody, 0)
    pltpu.sync_copy(stage_v.at[pl.ds(0, ZPT)], degs.at[pl.ds(row0, ZPT)])
    plsc.subcore_barrier()
    for c in range(2):
        wait_fetch(c)
        start_scatter(c)
    for c in range(2, 6):
        wait_fetch(c % 4)
        start_scatter(c % 4)
        wait_scatter((c + 2) % 4)
        start_fetch(c + 2, (c + 2) % 4)

    def group(g, carry):
        c0 = g * 4 + 6
        for k in range(4):
            p = (2 + k) % 4
            wait_fetch(p)
            start_scatter(p)
            wait_scatter(k % 4)
            start_fetch(c0 + k + 2, k % 4)
        return carry

    lax.fori_loop(0, (NCHUNK - 10) // 4, group, 0)
    for c in range(NCHUNK - 4, NCHUNK):
        p = c % 4
        wait_fetch(p)
        start_scatter(p)
        wait_scatter((c + 2) % 4)
        if c + 2 < NCHUNK:
            start_fetch(c + 2, (c + 2) % 4)
    wait_scatter((NCHUNK - 2) % 4)
    wait_scatter((NCHUNK - 1) % 4)
    pltpu.make_async_copy(dst_hbm.at[pl.ds(0, TAIL)], tidx, tsem).wait()
    pltpu.sync_copy(ones_v.at[pl.ds(0, TAIL)], degs.at[tidx], add=True)
    plsc.subcore_barrier()
    pltpu.sync_copy(degs.at[pl.ds(row0, ZPT)], stage_v.at[pl.ds(0, ZPT)])
    pltpu.sync_copy(stage_v.at[pl.ds(0, ZPT)],
                    deg_out.at[pl.ds(cid * ACC_ROWS + row0, ZPT)])


@functools.partial(
    pl.kernel,
    out_type=jax.ShapeDtypeStruct((NC, ACC_ROWS, D), jnp.float32),
    mesh=_mesh,
    scratch_types=[
        [pltpu.VMEM((CHUNK,), jnp.int32)] * 4,
        [pltpu.VMEM((CHUNK,), jnp.int32)] * 4,
        pltpu.VMEM((TAIL,), jnp.int32),
        pltpu.VMEM((TAIL,), jnp.int32),
        [pltpu.VMEM((CHUNK, D), jnp.float32)] * 2,
        pltpu.VMEM((TAIL, D), jnp.float32),
        pltpu.VMEM_SHARED((ACC_ROWS, D), jnp.float32),
        [pltpu.SemaphoreType.DMA] * 2,
        [pltpu.SemaphoreType.DMA] * 2,
        [pltpu.SemaphoreType.DMA] * 4,
        pltpu.SemaphoreType.DMA,
    ],
)
def _edge_kernel(y_hbm, src_hbm, dst_hbm, out_hbm,
                 sidx, didx, tsidx, tdidx, bufs, tbuf, acc,
                 gsems, ssems, isems, tsem):
    cid = lax.axis_index("c")
    sid = lax.axis_index("s")
    w = sid * NC + cid
    base = w * EPT
    row0 = sid * ZPT
    buf0 = bufs[0]

    def start_fetch(c, p):
        pltpu.async_copy(src_hbm.at[pl.ds(base + c * CHUNK, CHUNK)],
                         sidx[p], isems[p])
        pltpu.async_copy(dst_hbm.at[pl.ds(base + c * CHUNK, CHUNK)],
                         didx[p], isems[p])

    def wait_fetch(p):
        pltpu.make_async_copy(src_hbm.at[pl.ds(0, CHUNK)], sidx[p],
                              isems[p]).wait()
        pltpu.make_async_copy(dst_hbm.at[pl.ds(0, CHUNK)], didx[p],
                              isems[p]).wait()

    def start_gather(b, p):
        pltpu.async_copy(y_hbm.at[sidx[p]], bufs[b], gsems[b])

    def wait_gather(b):
        pltpu.make_async_copy(y_hbm.at[sidx[0]], bufs[b], gsems[b]).wait()

    def start_scatter(b, p):
        pltpu.async_copy(bufs[b], acc.at[didx[p]], ssems[b], add=True)

    def wait_scatter(b):
        pltpu.make_async_copy(bufs[0], acc.at[didx[0]], ssems[b]).wait()

    def body(c, k, fetch=True):
        b = k % 2
        p = k % 4
        wait_gather(b)
        start_scatter(b, p)
        wait_scatter(1 - b)
        if fetch:
            start_fetch(c + 3, (k + 3) % 4)
        wait_fetch((k + 1) % 4)
        start_gather(1 - b, (k + 1) % 4)

    pltpu.async_copy(src_hbm.at[pl.ds(base + NCHUNK * CHUNK, TAIL)],
                     tsidx, tsem)
    pltpu.async_copy(dst_hbm.at[pl.ds(base + NCHUNK * CHUNK, TAIL)],
                     tdidx, tsem)
    for s in range(4):
        start_fetch(s, s)
    wait_fetch(0)
    start_gather(0, 0)
    zeros16 = jnp.zeros((16,), jnp.float32)
    buf1 = bufs[1]

    def zbody(r, carry):
        for j in range(D // 16):
            buf1[r, pl.ds(j * 16, 16)] = zeros16
        return carry

    lax.fori_loop(0, CHUNK, zbody, 0)
    _rem = ZPT % CHUNK
    for k in range(ZPT // CHUNK):
        pltpu.sync_copy(buf1, acc.at[pl.ds(row0 + k * CHUNK, CHUNK)])
    if _rem:
        pltpu.sync_copy(buf1.at[pl.ds(0, _rem)],
                        acc.at[pl.ds(row0 + (ZPT // CHUNK) * CHUNK, _rem)])
    plsc.subcore_barrier()
    wait_gather(0)
    start_scatter(0, 0)
    wait_fetch(1)
    start_gather(1, 1)
    wait_gather(1)
    start_scatter(1, 1)
    wait_scatter(0)
    start_fetch(4, 0)
    wait_fetch(2)
    start_gather(0, 2)

    def group(g, carry):
        c0 = g * 4 + 2
        for k in range(4):
            body(c0 + k, 2 + k)
        return carry

    lax.fori_loop(0, (NCHUNK - 10) // 4, group, 0)
    for kk in range(7):
        c = NCHUNK - 8 + kk
        body(c, c, fetch=(c + 3 < NCHUNK))
    wait_gather((NCHUNK - 1) % 2)
    start_scatter((NCHUNK - 1) % 2, (NCHUNK - 1) % 4)
    wait_scatter((NCHUNK - 2) % 2)
    wait_scatter((NCHUNK - 1) % 2)
    pltpu.make_async_copy(src_hbm.at[pl.ds(0, TAIL)], tsidx, tsem).wait()
    pltpu.make_async_copy(dst_hbm.at[pl.ds(0, TAIL)], tdidx, tsem).wait()
    pltpu.async_copy(y_hbm.at[tsidx], tbuf, gsems[0])
    pltpu.make_async_copy(y_hbm.at[tsidx], tbuf, gsems[0]).wait()
    pltpu.sync_copy(tbuf, acc.at[tdidx], add=True)
    plsc.subcore_barrier()
    for k in range(ZPT // CHUNK):
        pltpu.sync_copy(acc.at[pl.ds(row0 + k * CHUNK, CHUNK)], buf0)
        pltpu.sync_copy(buf0, out_hbm.at[cid, pl.ds(row0 + k * CHUNK, CHUNK)])
    if _rem:
        _off = row0 + (ZPT // CHUNK) * CHUNK
        pltpu.sync_copy(acc.at[pl.ds(_off, _rem)], buf0.at[pl.ds(0, _rem)])
        pltpu.sync_copy(buf0.at[pl.ds(0, _rem)],
                        out_hbm.at[cid, pl.ds(_off, _rem)])


_BLK = 1000


def _mm_body(x_ref, w_ref, ds_ref, y_ref, dinv_ref):
    dinv = lax.rsqrt(ds_ref[...] + 1.0)
    xw = jnp.dot(x_ref[...], w_ref[...], preferred_element_type=jnp.float32)
    y_ref[...] = xw * dinv
    dinv_ref[...] = dinv


def _fin_body(a0_ref, a1_ref, y_ref, dinv_ref, b_ref, g_ref, be_ref, o_ref):
    s = a0_ref[0] + a1_ref[0] + y_ref[...]
    pre = s * dinv_ref[...] + b_ref[...]
    mu = jnp.mean(pre, axis=-1, keepdims=True)
    ctr = pre - mu
    var = jnp.mean(ctr * ctr, axis=-1, keepdims=True)
    h = ctr * lax.rsqrt(var + 1e-5) * g_ref[...] + be_ref[...]
    o_ref[...] = jnp.maximum(h, 0.0)


def kernel(x, edge_index, W, b, ln_gamma, ln_beta):
    ei = edge_index.astype(jnp.int32)
    src1 = ei[0]
    dst1 = ei[1]

    deg_parts = _deg_kernel(dst1)
    degsum = (deg_parts[:N] + deg_parts[ACC_ROWS:ACC_ROWS + N]).reshape(N, 1)

    y, dinv = pl.pallas_call(
        _mm_body,
        grid=(N // _BLK,),
        in_specs=[
            pl.BlockSpec((_BLK, D), lambda i: (i, 0)),
            pl.BlockSpec((D, D), lambda i: (0, 0)),
            pl.BlockSpec((_BLK, 1), lambda i: (i, 0)),
        ],
        out_specs=[
            pl.BlockSpec((_BLK, D), lambda i: (i, 0)),
            pl.BlockSpec((_BLK, 1), lambda i: (i, 0)),
        ],
        out_shape=[
            jax.ShapeDtypeStruct((N, D), jnp.float32),
            jax.ShapeDtypeStruct((N, 1), jnp.float32),
        ],
    )(x, W, degsum)

    acc_parts = _edge_kernel(y, src1, dst1)

    out = pl.pallas_call(
        _fin_body,
        grid=(N // _BLK,),
        in_specs=[
            pl.BlockSpec((1, _BLK, D), lambda i: (0, i, 0)),
            pl.BlockSpec((1, _BLK, D), lambda i: (1, i, 0)),
            pl.BlockSpec((_BLK, D), lambda i: (i, 0)),
            pl.BlockSpec((_BLK, 1), lambda i: (i, 0)),
            pl.BlockSpec((1, D), lambda i: (0, 0)),
            pl.BlockSpec((1, D), lambda i: (0, 0)),
            pl.BlockSpec((1, D), lambda i: (0, 0)),
        ],
        out_specs=pl.BlockSpec((_BLK, D), lambda i: (i, 0)),
        out_shape=jax.ShapeDtypeStruct((N, D), jnp.float32),
    )(acc_parts, acc_parts, y, dinv,
      b.reshape(1, D), ln_gamma.reshape(1, D), ln_beta.reshape(1, D))
    return out
